# Initial kernel scaffold; baseline (speedup 1.0000x reference)
import jax
import jax.numpy as jnp
from jax.experimental import pallas as pl


def kernel(inputs, token_table, pos_table):
    raise NotImplementedError("write your pallas kernel here")


# speedup vs baseline: 2.0547x; 2.0547x over previous
import functools

import jax
import jax.numpy as jnp
from jax import lax
from jax.experimental import pallas as pl
from jax.experimental.pallas import tpu as pltpu
from jax.experimental.pallas import tpu_sc as plsc

VOCAB = 100000
MAXLEN = 200
EMBED = 64
BATCH = 1024

NC = 2
NS = 16
NW = NC * NS
LANES = 16

ROWS = BATCH * MAXLEN
R_PER_W = ROWS // NW
CHUNK = 128
N_CHUNKS = R_PER_W // CHUNK
Q = EMBED // LANES


def _emb_kernel(idx_hbm, tok_hbm, pos_hbm, out_hbm,
                idx_v, rows_v, pos_v, sem_g):
    wid = lax.axis_index("s") * NC + lax.axis_index("c")
    base = wid * R_PER_W

    pltpu.sync_copy(pos_hbm, pos_v)

    def chunk_body(c, carry):
        s = base + c * CHUNK
        pltpu.sync_copy(idx_hbm.at[pl.ds(s, CHUNK)], idx_v)
        pltpu.async_copy(tok_hbm.at[idx_v], rows_v, sem_g).wait()

        def row_body(r, carry2):
            prow = lax.rem(s + r, MAXLEN)
            for q in range(Q):
                plsc.addupdate(rows_v.at[r, pl.ds(q * LANES, LANES)],
                               pos_v[prow, pl.ds(q * LANES, LANES)])
            return carry2

        lax.fori_loop(0, CHUNK, row_body, 0)
        pltpu.sync_copy(rows_v, out_hbm.at[pl.ds(s, CHUNK)])
        return carry

    lax.fori_loop(0, N_CHUNKS, chunk_body, 0)


@jax.jit
def _run(idx_flat, token_table, pos_table):
    mesh = plsc.VectorSubcoreMesh(core_axis_name="c", subcore_axis_name="s")
    f = pl.kernel(
        _emb_kernel,
        out_type=jax.ShapeDtypeStruct((ROWS, EMBED), jnp.float32),
        mesh=mesh,
        scratch_types=[
            pltpu.VMEM((CHUNK,), jnp.int32),
            pltpu.VMEM((CHUNK, EMBED), jnp.float32),
            pltpu.VMEM((MAXLEN, EMBED), jnp.float32),
            pltpu.SemaphoreType.DMA,
        ],
        compiler_params=pltpu.CompilerParams(use_tc_tiling_on_sc=False),
    )
    return f(idx_flat, token_table, pos_table)


def kernel(inputs, token_table, pos_table):
    idx_flat = inputs.reshape(-1).astype(jnp.int32)
    out = _run(idx_flat, token_table, pos_table)
    return out.reshape(BATCH, MAXLEN, EMBED)


# speedup vs baseline: 3.0727x; 1.4955x over previous
import jax
import jax.numpy as jnp
from jax import lax
from jax.experimental import pallas as pl
from jax.experimental.pallas import tpu as pltpu
from jax.experimental.pallas import tpu_sc as plsc

VOCAB = 100000
MAXLEN = 200
EMBED = 64
BATCH = 1024

NC = 2
NS = 16
NW = NC * NS
LANES = 16

ROWS = BATCH * MAXLEN
R_PER_W = ROWS // NW
CHUNK = MAXLEN
N_CHUNKS = R_PER_W // CHUNK
Q = EMBED // LANES
NB = 3


def _emb_kernel(idx_hbm, tok_hbm, pos_hbm, out_hbm,
                idx_v, rows_v, pos_v, sg0, sg1, sg2, so0, so1, so2):
    semg = (sg0, sg1, sg2)
    semo = (so0, so1, so2)
    wid = lax.axis_index("s") * NC + lax.axis_index("c")
    base = wid * R_PER_W

    pltpu.sync_copy(pos_hbm, pos_v)

    def start_gather(c):
        b = c % NB
        s = base + c * CHUNK
        pltpu.sync_copy(idx_hbm.at[pl.ds(s, CHUNK)], idx_v.at[b])
        return pltpu.async_copy(tok_hbm.at[idx_v.at[b]], rows_v.at[b], semg[b])

    pending_g = {0: start_gather(0)}
    pending_o = {}
    for c in range(N_CHUNKS):
        b = c % NB
        nxt = c + 1
        if nxt < N_CHUNKS:
            if nxt - NB >= 0:
                pending_o.pop(nxt - NB).wait()
            pending_g[nxt] = start_gather(nxt)
        pending_g.pop(c).wait()

        def row_body(r, b=b):
            for q in range(Q):
                plsc.addupdate(rows_v.at[b, r, pl.ds(q * LANES, LANES)],
                               pos_v[r, pl.ds(q * LANES, LANES)])

        lax.fori_loop(0, CHUNK, lambda r, car, b=b: (row_body(r, b=b), car)[1], 0)

        s = base + c * CHUNK
        pending_o[c] = pltpu.async_copy(
            rows_v.at[b], out_hbm.at[pl.ds(s, CHUNK)], semo[b])

    for c in sorted(pending_o):
        pending_o.pop(c).wait()


@jax.jit
def _run(idx_flat, token_table, pos_table):
    mesh = plsc.VectorSubcoreMesh(core_axis_name="c", subcore_axis_name="s")
    f = pl.kernel(
        _emb_kernel,
        out_type=jax.ShapeDtypeStruct((ROWS, EMBED), jnp.float32),
        mesh=mesh,
        scratch_types=[
            pltpu.VMEM((NB, CHUNK), jnp.int32),
            pltpu.VMEM((NB, CHUNK, EMBED), jnp.float32),
            pltpu.VMEM((MAXLEN, EMBED), jnp.float32),
        ] + [pltpu.SemaphoreType.DMA] * (2 * NB),
        compiler_params=pltpu.CompilerParams(use_tc_tiling_on_sc=False),
    )
    return f(idx_flat, token_table, pos_table)


def kernel(inputs, token_table, pos_table):
    idx_flat = inputs.reshape(-1).astype(jnp.int32)
    out = _run(idx_flat, token_table, pos_table)
    return out.reshape(BATCH, MAXLEN, EMBED)
